# baseline (speedup 1.0000x reference)
import math

import jax
import jax.numpy as jnp
from jax.experimental import pallas as pl

N = 10000
NP = 10240
D = 64
FD = 128
K = 16
B = 2
RB = 256

_INV_BN = 1.0 / math.sqrt(1.0 + 1e-5)
_NEG_PAD = -2.0
_NEG_TAKEN = -3.0


def _prep_body(emb_ref, x_ref, wlin_ref, atti_ref, attj_ref, attemi_ref,
               attemj_ref, embn_ref, xl_ref, si_ref, sj_ref):
    emb = emb_ref[...]
    sumsq = jnp.sum(emb * emb, axis=1, keepdims=True)
    inv = jnp.where(sumsq > 0.0, 1.0 / jnp.sqrt(sumsq), 0.0)
    embn_ref[...] = emb * inv
    e_i = jnp.sum(emb * attemi_ref[0][None, :], axis=1, keepdims=True)
    e_j = jax.lax.dot_general(attemj_ref[...], emb, (((1,), (1,)), ((), ())),
                              preferred_element_type=jnp.float32,
                              precision=jax.lax.Precision.HIGHEST)
    for b in range(B):
        xl = jax.lax.dot_general(x_ref[b], wlin_ref[...],
                                 (((1,), (1,)), ((), ())),
                                 preferred_element_type=jnp.float32,
                                 precision=jax.lax.Precision.HIGHEST)
        xl_ref[b] = xl
        si_ref[:, b:b + 1] = (
            jnp.sum(xl * atti_ref[0][None, :], axis=1, keepdims=True) + e_i)
        sj_ref[b:b + 1, :] = (
            jax.lax.dot_general(attj_ref[...], xl, (((1,), (1,)), ((), ())),
                                preferred_element_type=jnp.float32,
                                precision=jax.lax.Precision.HIGHEST) + e_j)


def _main_body(embn_blk_ref, embn_ref, xl_ref, si_ref, sj_ref, emb_blk_ref,
               gbias_ref, g1_ref, b1_ref, g2_ref, b2_ref, wout_ref, bout_ref,
               y_ref):
    s = jax.lax.dot_general(embn_blk_ref[...], embn_ref[...],
                            (((1,), (1,)), ((), ())),
                            preferred_element_type=jnp.float32,
                            precision=jax.lax.Precision.HIGHEST)
    lane = jax.lax.broadcasted_iota(jnp.int32, (RB, NP), 1)
    s = jnp.where(lane < N, s, _NEG_PAD)
    NC = NP // 128
    sc = s.reshape(RB, NC, 128)
    cmax = jnp.max(sc, axis=2)
    iota_c = jax.lax.broadcasted_iota(jnp.int32, (RB, NC), 1)
    workc = cmax
    ohs = []
    for r in range(K):
        mc = jnp.max(workc, axis=1, keepdims=True)
        eq = workc == mc
        pick = jnp.max(jnp.where(eq, iota_c, -1), axis=1, keepdims=True)
        first = iota_c == pick
        ohs.append(first.astype(jnp.float32)[:, None, :])
        workc = jnp.where(first, _NEG_TAKEN, workc)
    oh = jnp.concatenate(ohs, axis=1)
    cand = jax.lax.dot_general(
        oh, sc, (((2,), (1,)), ((0,), (0,))),
        preferred_element_type=jnp.float32,
        precision=jax.lax.Precision.HIGHEST)
    work = cand.reshape(RB, K * 128)
    m = None
    for r in range(K):
        m = jnp.max(work, axis=1, keepdims=True)
        if r < K - 1:
            work = jnp.where(work == m, _NEG_TAKEN, work)
    sel = s >= m
    emb_blk = emb_blk_ref[...]
    for b in range(B):
        alpha = si_ref[:, b:b + 1] + sj_ref[b:b + 1, :]
        alpha = jnp.where(alpha >= 0.0, alpha, 0.2 * alpha)
        e = jnp.where(sel, jnp.exp(alpha), 0.0)
        denom = jnp.sum(e, axis=1, keepdims=True) + 1e-16
        agg = jax.lax.dot_general(e, xl_ref[b], (((1,), (0,)), ((), ())),
                                  preferred_element_type=jnp.float32)
        agg = agg / denom + gbias_ref[0][None, :]
        h = g1_ref[0][None, :] * agg * _INV_BN + b1_ref[0][None, :]
        h = jnp.maximum(h, 0.0)
        h = h * emb_blk
        h = g2_ref[0][None, :] * h * _INV_BN + b2_ref[0][None, :]
        h = jnp.maximum(h, 0.0)
        y = jnp.sum(h * wout_ref[0][None, :], axis=1, keepdims=True)
        y_ref[:, b:b + 1] = y + bout_ref[0][0]


def kernel(data, org_edge_index, labels, emb, W_lin, att_i, att_j, att_em_i,
           att_em_j, gnn_bias, bn1_gamma, bn1_beta, bn2_gamma, bn2_beta,
           W_out, b_out):
    del org_edge_index, labels
    pad_n = NP - N
    emb_p = jnp.pad(emb, ((0, pad_n), (0, 0)))
    x_p = jnp.pad(data, ((0, 0), (0, pad_n), (0, 0)))
    r1 = lambda v: v.reshape(1, D)

    PB = 2048
    embn, xl, si, sj = pl.pallas_call(
        _prep_body,
        grid=(NP // PB,),
        in_specs=[
            pl.BlockSpec((PB, D), lambda i: (i, 0)),
            pl.BlockSpec((B, PB, FD), lambda i: (0, i, 0)),
            pl.BlockSpec((D, FD), lambda i: (0, 0)),
            pl.BlockSpec((1, D), lambda i: (0, 0)),
            pl.BlockSpec((1, D), lambda i: (0, 0)),
            pl.BlockSpec((1, D), lambda i: (0, 0)),
            pl.BlockSpec((1, D), lambda i: (0, 0)),
        ],
        out_specs=(
            pl.BlockSpec((PB, D), lambda i: (i, 0)),
            pl.BlockSpec((B, PB, D), lambda i: (0, i, 0)),
            pl.BlockSpec((PB, B), lambda i: (i, 0)),
            pl.BlockSpec((B, PB), lambda i: (0, i)),
        ),
        out_shape=(
            jax.ShapeDtypeStruct((NP, D), jnp.float32),
            jax.ShapeDtypeStruct((B, NP, D), jnp.float32),
            jax.ShapeDtypeStruct((NP, B), jnp.float32),
            jax.ShapeDtypeStruct((B, NP), jnp.float32),
        ),
    )(emb_p, x_p, W_lin, r1(att_i), r1(att_j), r1(att_em_i), r1(att_em_j))

    y = pl.pallas_call(
        _main_body,
        grid=(NP // RB,),
        in_specs=[
            pl.BlockSpec((RB, D), lambda i: (i, 0)),
            pl.BlockSpec((NP, D), lambda i: (0, 0)),
            pl.BlockSpec((B, NP, D), lambda i: (0, 0, 0)),
            pl.BlockSpec((RB, B), lambda i: (i, 0)),
            pl.BlockSpec((B, NP), lambda i: (0, 0)),
            pl.BlockSpec((RB, D), lambda i: (i, 0)),
            pl.BlockSpec((1, D), lambda i: (0, 0)),
            pl.BlockSpec((1, D), lambda i: (0, 0)),
            pl.BlockSpec((1, D), lambda i: (0, 0)),
            pl.BlockSpec((1, D), lambda i: (0, 0)),
            pl.BlockSpec((1, D), lambda i: (0, 0)),
            pl.BlockSpec((1, D), lambda i: (0, 0)),
            pl.BlockSpec((1, 1), lambda i: (0, 0)),
        ],
        out_specs=pl.BlockSpec((RB, B), lambda i: (i, 0)),
        out_shape=jax.ShapeDtypeStruct((NP, B), jnp.float32),
    )(embn, embn, xl, si, sj, emb_p, r1(gnn_bias), r1(bn1_gamma),
      r1(bn1_beta), r1(bn2_gamma), r1(bn2_beta), W_out.reshape(1, D),
      b_out.reshape(1, 1))

    return y[:N, :].T


# speedup vs baseline: 1.6378x; 1.6378x over previous
import math

import jax
import jax.numpy as jnp
from jax.experimental import pallas as pl

N = 10000
NP = 10240
D = 64
FD = 128
K = 16
B = 2
RB = 256

_INV_BN = 1.0 / math.sqrt(1.0 + 1e-5)
_NEG_PAD = -2.0
_NEG_TAKEN = -3.0


def _prep_body(emb_ref, x_ref, wlin_ref, atti_ref, attj_ref, attemi_ref,
               attemj_ref, embn_ref, xl_ref, si_ref, sj_ref):
    emb = emb_ref[...]
    sumsq = jnp.sum(emb * emb, axis=1, keepdims=True)
    inv = jnp.where(sumsq > 0.0, 1.0 / jnp.sqrt(sumsq), 0.0)
    embn_ref[...] = emb * inv
    e_i = jnp.sum(emb * attemi_ref[0][None, :], axis=1, keepdims=True)
    e_j = jax.lax.dot_general(attemj_ref[...], emb, (((1,), (1,)), ((), ())),
                              preferred_element_type=jnp.float32,
                              precision=jax.lax.Precision.HIGHEST)
    for b in range(B):
        xl = jax.lax.dot_general(x_ref[b], wlin_ref[...],
                                 (((1,), (1,)), ((), ())),
                                 preferred_element_type=jnp.float32,
                                 precision=jax.lax.Precision.HIGHEST)
        xl_ref[b] = xl
        si_ref[:, b:b + 1] = (
            jnp.sum(xl * atti_ref[0][None, :], axis=1, keepdims=True) + e_i)
        sj_ref[b:b + 1, :] = (
            jax.lax.dot_general(attj_ref[...], xl, (((1,), (1,)), ((), ())),
                                preferred_element_type=jnp.float32,
                                precision=jax.lax.Precision.HIGHEST) + e_j)


def _main_body(embn_blk_ref, embn_ref, xl_ref, si_ref, sj_ref, emb_blk_ref,
               gbias_ref, g1_ref, b1_ref, g2_ref, b2_ref, wout_ref, bout_ref,
               y_ref):
    s = jax.lax.dot_general(embn_blk_ref[...], embn_ref[...],
                            (((1,), (1,)), ((), ())),
                            preferred_element_type=jnp.float32,
                            precision=jax.lax.Precision.HIGHEST)
    lane = jax.lax.broadcasted_iota(jnp.int32, (RB, NP), 1)
    s = jnp.where(lane < N, s, _NEG_PAD)
    work = s
    m = None
    for r in range(K):
        m = jnp.max(work, axis=1, keepdims=True)
        if r < K - 1:
            work = jnp.where(work == m, _NEG_TAKEN, work)
    sel = s >= m
    emb_blk = emb_blk_ref[...]
    for b in range(B):
        alpha = si_ref[:, b:b + 1] + sj_ref[b:b + 1, :]
        alpha = jnp.where(alpha >= 0.0, alpha, 0.2 * alpha)
        e = jnp.where(sel, jnp.exp(alpha), 0.0)
        denom = jnp.sum(e, axis=1, keepdims=True) + 1e-16
        agg = jax.lax.dot_general(e, xl_ref[b], (((1,), (0,)), ((), ())),
                                  preferred_element_type=jnp.float32)
        agg = agg / denom + gbias_ref[0][None, :]
        h = g1_ref[0][None, :] * agg * _INV_BN + b1_ref[0][None, :]
        h = jnp.maximum(h, 0.0)
        h = h * emb_blk
        h = g2_ref[0][None, :] * h * _INV_BN + b2_ref[0][None, :]
        h = jnp.maximum(h, 0.0)
        y = jnp.sum(h * wout_ref[0][None, :], axis=1, keepdims=True)
        y_ref[:, b:b + 1] = y + bout_ref[0][0]


def kernel(data, org_edge_index, labels, emb, W_lin, att_i, att_j, att_em_i,
           att_em_j, gnn_bias, bn1_gamma, bn1_beta, bn2_gamma, bn2_beta,
           W_out, b_out):
    del org_edge_index, labels
    pad_n = NP - N
    emb_p = jnp.pad(emb, ((0, pad_n), (0, 0)))
    x_p = jnp.pad(data, ((0, 0), (0, pad_n), (0, 0)))
    r1 = lambda v: v.reshape(1, D)

    PB = 2048
    embn, xl, si, sj = pl.pallas_call(
        _prep_body,
        grid=(NP // PB,),
        in_specs=[
            pl.BlockSpec((PB, D), lambda i: (i, 0)),
            pl.BlockSpec((B, PB, FD), lambda i: (0, i, 0)),
            pl.BlockSpec((D, FD), lambda i: (0, 0)),
            pl.BlockSpec((1, D), lambda i: (0, 0)),
            pl.BlockSpec((1, D), lambda i: (0, 0)),
            pl.BlockSpec((1, D), lambda i: (0, 0)),
            pl.BlockSpec((1, D), lambda i: (0, 0)),
        ],
        out_specs=(
            pl.BlockSpec((PB, D), lambda i: (i, 0)),
            pl.BlockSpec((B, PB, D), lambda i: (0, i, 0)),
            pl.BlockSpec((PB, B), lambda i: (i, 0)),
            pl.BlockSpec((B, PB), lambda i: (0, i)),
        ),
        out_shape=(
            jax.ShapeDtypeStruct((NP, D), jnp.float32),
            jax.ShapeDtypeStruct((B, NP, D), jnp.float32),
            jax.ShapeDtypeStruct((NP, B), jnp.float32),
            jax.ShapeDtypeStruct((B, NP), jnp.float32),
        ),
    )(emb_p, x_p, W_lin, r1(att_i), r1(att_j), r1(att_em_i), r1(att_em_j))

    y = pl.pallas_call(
        _main_body,
        grid=(NP // RB,),
        in_specs=[
            pl.BlockSpec((RB, D), lambda i: (i, 0)),
            pl.BlockSpec((NP, D), lambda i: (0, 0)),
            pl.BlockSpec((B, NP, D), lambda i: (0, 0, 0)),
            pl.BlockSpec((RB, B), lambda i: (i, 0)),
            pl.BlockSpec((B, NP), lambda i: (0, 0)),
            pl.BlockSpec((RB, D), lambda i: (i, 0)),
            pl.BlockSpec((1, D), lambda i: (0, 0)),
            pl.BlockSpec((1, D), lambda i: (0, 0)),
            pl.BlockSpec((1, D), lambda i: (0, 0)),
            pl.BlockSpec((1, D), lambda i: (0, 0)),
            pl.BlockSpec((1, D), lambda i: (0, 0)),
            pl.BlockSpec((1, D), lambda i: (0, 0)),
            pl.BlockSpec((1, 1), lambda i: (0, 0)),
        ],
        out_specs=pl.BlockSpec((RB, B), lambda i: (i, 0)),
        out_shape=jax.ShapeDtypeStruct((NP, B), jnp.float32),
    )(embn, embn, xl, si, sj, emb_p, r1(gnn_bias), r1(bn1_gamma),
      r1(bn1_beta), r1(bn2_gamma), r1(bn2_beta), W_out.reshape(1, D),
      b_out.reshape(1, 1))

    return y[:N, :].T


# speedup vs baseline: 1.6501x; 1.0075x over previous
import math

import jax
import jax.numpy as jnp
from jax.experimental import pallas as pl

N = 10000
NP = 10240
D = 64
FD = 128
K = 16
B = 2
RB = 256

_INV_BN = 1.0 / math.sqrt(1.0 + 1e-5)
_NEG_PAD = -2.0
_NEG_TAKEN = -3.0


def _prep_body(emb_ref, x_ref, wlin_ref, atti_ref, attj_ref, attemi_ref,
               attemj_ref, embn_ref, xl_ref, si_ref, sj_ref):
    emb = emb_ref[...]
    sumsq = jnp.sum(emb * emb, axis=1, keepdims=True)
    inv = jnp.where(sumsq > 0.0, 1.0 / jnp.sqrt(sumsq), 0.0)
    embn_ref[...] = emb * inv
    e_i = jnp.sum(emb * attemi_ref[0][None, :], axis=1, keepdims=True)
    e_j = jax.lax.dot_general(attemj_ref[...], emb, (((1,), (1,)), ((), ())),
                              preferred_element_type=jnp.float32,
                              precision=jax.lax.Precision.HIGHEST)
    for b in range(B):
        xl = jax.lax.dot_general(x_ref[b], wlin_ref[...],
                                 (((1,), (1,)), ((), ())),
                                 preferred_element_type=jnp.float32,
                                 precision=jax.lax.Precision.HIGHEST)
        xl_ref[b] = xl
        si_ref[:, b:b + 1] = (
            jnp.sum(xl * atti_ref[0][None, :], axis=1, keepdims=True) + e_i)
        sj_ref[b:b + 1, :] = (
            jax.lax.dot_general(attj_ref[...], xl, (((1,), (1,)), ((), ())),
                                preferred_element_type=jnp.float32,
                                precision=jax.lax.Precision.HIGHEST) + e_j)


def _main_body(embn_blk_ref, embn_ref, xl_ref, si_ref, sj_ref, emb_blk_ref,
               gbias_ref, g1_ref, b1_ref, g2_ref, b2_ref, wout_ref, bout_ref,
               y_ref):
    s = jax.lax.dot_general(embn_blk_ref[...], embn_ref[...],
                            (((1,), (1,)), ((), ())),
                            preferred_element_type=jnp.float32,
                            precision=jax.lax.Precision.HIGHEST)
    lane = jax.lax.broadcasted_iota(jnp.int32, (RB, NP), 1)
    s = jnp.where(lane < N, s, _NEG_PAD)
    m = jnp.max(s, axis=1, keepdims=True)
    for r in range(K - 1):
        m = jnp.max(jnp.where(s < m, s, _NEG_TAKEN), axis=1, keepdims=True)
    sel = s >= m
    emb_blk = emb_blk_ref[...]
    for b in range(B):
        alpha = si_ref[:, b:b + 1] + sj_ref[b:b + 1, :]
        alpha = jnp.where(alpha >= 0.0, alpha, 0.2 * alpha)
        e = jnp.where(sel, jnp.exp(alpha), 0.0)
        denom = jnp.sum(e, axis=1, keepdims=True) + 1e-16
        agg = jax.lax.dot_general(e, xl_ref[b], (((1,), (0,)), ((), ())),
                                  preferred_element_type=jnp.float32)
        agg = agg / denom + gbias_ref[0][None, :]
        h = g1_ref[0][None, :] * agg * _INV_BN + b1_ref[0][None, :]
        h = jnp.maximum(h, 0.0)
        h = h * emb_blk
        h = g2_ref[0][None, :] * h * _INV_BN + b2_ref[0][None, :]
        h = jnp.maximum(h, 0.0)
        y = jnp.sum(h * wout_ref[0][None, :], axis=1, keepdims=True)
        y_ref[:, b:b + 1] = y + bout_ref[0][0]


def kernel(data, org_edge_index, labels, emb, W_lin, att_i, att_j, att_em_i,
           att_em_j, gnn_bias, bn1_gamma, bn1_beta, bn2_gamma, bn2_beta,
           W_out, b_out):
    del org_edge_index, labels
    pad_n = NP - N
    emb_p = jnp.pad(emb, ((0, pad_n), (0, 0)))
    x_p = jnp.pad(data, ((0, 0), (0, pad_n), (0, 0)))
    r1 = lambda v: v.reshape(1, D)

    PB = 2048
    embn, xl, si, sj = pl.pallas_call(
        _prep_body,
        grid=(NP // PB,),
        in_specs=[
            pl.BlockSpec((PB, D), lambda i: (i, 0)),
            pl.BlockSpec((B, PB, FD), lambda i: (0, i, 0)),
            pl.BlockSpec((D, FD), lambda i: (0, 0)),
            pl.BlockSpec((1, D), lambda i: (0, 0)),
            pl.BlockSpec((1, D), lambda i: (0, 0)),
            pl.BlockSpec((1, D), lambda i: (0, 0)),
            pl.BlockSpec((1, D), lambda i: (0, 0)),
        ],
        out_specs=(
            pl.BlockSpec((PB, D), lambda i: (i, 0)),
            pl.BlockSpec((B, PB, D), lambda i: (0, i, 0)),
            pl.BlockSpec((PB, B), lambda i: (i, 0)),
            pl.BlockSpec((B, PB), lambda i: (0, i)),
        ),
        out_shape=(
            jax.ShapeDtypeStruct((NP, D), jnp.float32),
            jax.ShapeDtypeStruct((B, NP, D), jnp.float32),
            jax.ShapeDtypeStruct((NP, B), jnp.float32),
            jax.ShapeDtypeStruct((B, NP), jnp.float32),
        ),
    )(emb_p, x_p, W_lin, r1(att_i), r1(att_j), r1(att_em_i), r1(att_em_j))

    y = pl.pallas_call(
        _main_body,
        grid=(NP // RB,),
        in_specs=[
            pl.BlockSpec((RB, D), lambda i: (i, 0)),
            pl.BlockSpec((NP, D), lambda i: (0, 0)),
            pl.BlockSpec((B, NP, D), lambda i: (0, 0, 0)),
            pl.BlockSpec((RB, B), lambda i: (i, 0)),
            pl.BlockSpec((B, NP), lambda i: (0, 0)),
            pl.BlockSpec((RB, D), lambda i: (i, 0)),
            pl.BlockSpec((1, D), lambda i: (0, 0)),
            pl.BlockSpec((1, D), lambda i: (0, 0)),
            pl.BlockSpec((1, D), lambda i: (0, 0)),
            pl.BlockSpec((1, D), lambda i: (0, 0)),
            pl.BlockSpec((1, D), lambda i: (0, 0)),
            pl.BlockSpec((1, D), lambda i: (0, 0)),
            pl.BlockSpec((1, 1), lambda i: (0, 0)),
        ],
        out_specs=pl.BlockSpec((RB, B), lambda i: (i, 0)),
        out_shape=jax.ShapeDtypeStruct((NP, B), jnp.float32),
    )(embn, embn, xl, si, sj, emb_p, r1(gnn_bias), r1(bn1_gamma),
      r1(bn1_beta), r1(bn2_gamma), r1(bn2_beta), W_out.reshape(1, D),
      b_out.reshape(1, 1))

    return y[:N, :].T


# speedup vs baseline: 1.7914x; 1.0856x over previous
import math

import jax
import jax.numpy as jnp
from jax.experimental import pallas as pl

N = 10000
NP = 10240
D = 64
FD = 128
K = 16
B = 2
RB = 256

_INV_BN = 1.0 / math.sqrt(1.0 + 1e-5)
_NEG_PAD = -2.0
_NEG_TAKEN = -3.0


def _prep_body(emb_ref, x_ref, wlin_ref, atti_ref, attj_ref, attemi_ref,
               attemj_ref, embn_ref, xl_ref, si_ref, sj_ref):
    emb = emb_ref[...]
    sumsq = jnp.sum(emb * emb, axis=1, keepdims=True)
    inv = jnp.where(sumsq > 0.0, 1.0 / jnp.sqrt(sumsq), 0.0)
    embn_ref[...] = emb * inv
    e_i = jnp.sum(emb * attemi_ref[0][None, :], axis=1, keepdims=True)
    e_j = jax.lax.dot_general(attemj_ref[...], emb, (((1,), (1,)), ((), ())),
                              preferred_element_type=jnp.float32,
                              precision=jax.lax.Precision.HIGHEST)
    for b in range(B):
        xl = jax.lax.dot_general(x_ref[b], wlin_ref[...],
                                 (((1,), (1,)), ((), ())),
                                 preferred_element_type=jnp.float32,
                                 precision=jax.lax.Precision.HIGHEST)
        xl_ref[b] = xl
        si_ref[:, b:b + 1] = (
            jnp.sum(xl * atti_ref[0][None, :], axis=1, keepdims=True) + e_i)
        sj_ref[b:b + 1, :] = (
            jax.lax.dot_general(attj_ref[...], xl, (((1,), (1,)), ((), ())),
                                preferred_element_type=jnp.float32,
                                precision=jax.lax.Precision.HIGHEST) + e_j)


def _main_body(embn_blk_ref, embn_ref, xl_ref, si_ref, sj_ref, emb_blk_ref,
               gbias_ref, g1_ref, b1_ref, g2_ref, b2_ref, wout_ref, bout_ref,
               y_ref):
    s = jax.lax.dot_general(embn_blk_ref[...], embn_ref[...],
                            (((1,), (1,)), ((), ())),
                            preferred_element_type=jnp.float32,
                            precision=jax.lax.Precision.HIGHEST)
    lane = jax.lax.broadcasted_iota(jnp.int32, (RB, NP), 1)
    s = jnp.where(lane < N, s, _NEG_PAD)
    NC = NP // 128
    sc = s.reshape(RB, NC, 128)
    fold = jnp.max(sc, axis=1)
    iota_l = jax.lax.broadcasted_iota(jnp.int32, (RB, 128), 1)
    workl = fold
    lpicks = []
    for r in range(K):
        ml = jnp.max(workl, axis=1, keepdims=True)
        eq = workl == ml
        pick = jnp.max(jnp.where(eq, iota_l, -1), axis=1, keepdims=True)
        lpicks.append(pick)
        workl = jnp.where(iota_l == pick, _NEG_TAKEN, workl)
    lane_ids = jnp.concatenate(lpicks, axis=1)
    idx3 = jnp.broadcast_to(lane_ids[:, None, :], (RB, NC, K))
    cand = jnp.take_along_axis(sc, idx3, axis=2,
                               mode="promise_in_bounds")
    candf = cand.reshape(RB, NC * K)
    m = jnp.max(candf, axis=1, keepdims=True)
    for r in range(K - 1):
        m = jnp.max(jnp.where(candf < m, candf, _NEG_TAKEN), axis=1,
                    keepdims=True)
    sel = s >= m
    emb_blk = emb_blk_ref[...]
    for b in range(B):
        alpha = si_ref[:, b:b + 1] + sj_ref[b:b + 1, :]
        alpha = jnp.where(alpha >= 0.0, alpha, 0.2 * alpha)
        e = jnp.where(sel, jnp.exp(alpha), 0.0)
        denom = jnp.sum(e, axis=1, keepdims=True) + 1e-16
        agg = jax.lax.dot_general(e, xl_ref[b], (((1,), (0,)), ((), ())),
                                  preferred_element_type=jnp.float32)
        agg = agg / denom + gbias_ref[0][None, :]
        h = g1_ref[0][None, :] * agg * _INV_BN + b1_ref[0][None, :]
        h = jnp.maximum(h, 0.0)
        h = h * emb_blk
        h = g2_ref[0][None, :] * h * _INV_BN + b2_ref[0][None, :]
        h = jnp.maximum(h, 0.0)
        y = jnp.sum(h * wout_ref[0][None, :], axis=1, keepdims=True)
        y_ref[:, b:b + 1] = y + bout_ref[0][0]


def kernel(data, org_edge_index, labels, emb, W_lin, att_i, att_j, att_em_i,
           att_em_j, gnn_bias, bn1_gamma, bn1_beta, bn2_gamma, bn2_beta,
           W_out, b_out):
    del org_edge_index, labels
    pad_n = NP - N
    emb_p = jnp.pad(emb, ((0, pad_n), (0, 0)))
    x_p = jnp.pad(data, ((0, 0), (0, pad_n), (0, 0)))
    r1 = lambda v: v.reshape(1, D)

    PB = 2048
    embn, xl, si, sj = pl.pallas_call(
        _prep_body,
        grid=(NP // PB,),
        in_specs=[
            pl.BlockSpec((PB, D), lambda i: (i, 0)),
            pl.BlockSpec((B, PB, FD), lambda i: (0, i, 0)),
            pl.BlockSpec((D, FD), lambda i: (0, 0)),
            pl.BlockSpec((1, D), lambda i: (0, 0)),
            pl.BlockSpec((1, D), lambda i: (0, 0)),
            pl.BlockSpec((1, D), lambda i: (0, 0)),
            pl.BlockSpec((1, D), lambda i: (0, 0)),
        ],
        out_specs=(
            pl.BlockSpec((PB, D), lambda i: (i, 0)),
            pl.BlockSpec((B, PB, D), lambda i: (0, i, 0)),
            pl.BlockSpec((PB, B), lambda i: (i, 0)),
            pl.BlockSpec((B, PB), lambda i: (0, i)),
        ),
        out_shape=(
            jax.ShapeDtypeStruct((NP, D), jnp.float32),
            jax.ShapeDtypeStruct((B, NP, D), jnp.float32),
            jax.ShapeDtypeStruct((NP, B), jnp.float32),
            jax.ShapeDtypeStruct((B, NP), jnp.float32),
        ),
    )(emb_p, x_p, W_lin, r1(att_i), r1(att_j), r1(att_em_i), r1(att_em_j))

    y = pl.pallas_call(
        _main_body,
        grid=(NP // RB,),
        in_specs=[
            pl.BlockSpec((RB, D), lambda i: (i, 0)),
            pl.BlockSpec((NP, D), lambda i: (0, 0)),
            pl.BlockSpec((B, NP, D), lambda i: (0, 0, 0)),
            pl.BlockSpec((RB, B), lambda i: (i, 0)),
            pl.BlockSpec((B, NP), lambda i: (0, 0)),
            pl.BlockSpec((RB, D), lambda i: (i, 0)),
            pl.BlockSpec((1, D), lambda i: (0, 0)),
            pl.BlockSpec((1, D), lambda i: (0, 0)),
            pl.BlockSpec((1, D), lambda i: (0, 0)),
            pl.BlockSpec((1, D), lambda i: (0, 0)),
            pl.BlockSpec((1, D), lambda i: (0, 0)),
            pl.BlockSpec((1, D), lambda i: (0, 0)),
            pl.BlockSpec((1, 1), lambda i: (0, 0)),
        ],
        out_specs=pl.BlockSpec((RB, B), lambda i: (i, 0)),
        out_shape=jax.ShapeDtypeStruct((NP, B), jnp.float32),
    )(embn, embn, xl, si, sj, emb_p, r1(gnn_bias), r1(bn1_gamma),
      r1(bn1_beta), r1(bn2_gamma), r1(bn2_beta), W_out.reshape(1, D),
      b_out.reshape(1, 1))

    return y[:N, :].T
